# baseline (speedup 1.0000x reference)
import functools

import jax
import jax.numpy as jnp
from jax import lax
from jax.experimental import pallas as pl
from jax.experimental.pallas import tpu as pltpu
from jax.experimental.pallas import tpu_sc as plsc

D = 128
NP = 10240
NC = 2
NS = 16
L = 16
CK = 64
GRP = 8
NBUF = 4
BLK = 128


def _mesh():
    return plsc.VectorSubcoreMesh(
        core_axis_name="c", subcore_axis_name="s", num_cores=NC, num_subcores=NS
    )


@functools.cache
def _deg_kernel(ngroups):
    G = ngroups // (NC * NS)

    @functools.partial(
        pl.kernel,
        mesh=_mesh(),
        compiler_params=pltpu.CompilerParams(needs_layout_passes=False),
        out_type=jax.ShapeDtypeStruct((NC, NP), jnp.float32),
        scratch_types=[
            pltpu.VMEM((G, GRP, CK), jnp.int32),
            pltpu.VMEM((G, GRP, CK), jnp.float32),
            pltpu.VMEM((NP // NS,), jnp.float32),
            pltpu.VMEM_SHARED((NP,), jnp.float32),
            pltpu.SemaphoreType.DMA,
        ],
    )
    def k(dst3, ew3, deg, dstb3, ewb3, zbuf, deg_sp, ssem):
        c = lax.axis_index("c")
        s = lax.axis_index("s")
        w = c * NS + s
        for j in range(0, NP // NS, L):
            zbuf[pl.ds(j, L)] = jnp.zeros((L,), jnp.float32)
        pltpu.sync_copy(zbuf, deg_sp.at[pl.ds(s * (NP // NS), NP // NS)])
        pltpu.sync_copy(dst3.at[pl.ds(w * G, G)], dstb3)
        pltpu.sync_copy(ew3.at[pl.ds(w * G, G)], ewb3)
        plsc.subcore_barrier()

        def grp_body(g, carry):
            for j in range(GRP):
                pltpu.async_copy(ewb3.at[g, j], deg_sp.at[dstb3.at[g, j]],
                                 ssem, add=True)
            for j in range(GRP):
                pltpu.make_async_copy(ewb3.at[g, j],
                                      deg_sp.at[dstb3.at[g, j]], ssem).wait()
            return carry

        lax.fori_loop(0, G, grp_body, 0)
        plsc.subcore_barrier()
        pltpu.sync_copy(deg_sp.at[pl.ds(s * (NP // NS), NP // NS)], zbuf)
        pltpu.sync_copy(zbuf, deg.at[c, pl.ds(s * (NP // NS), NP // NS)])

    return k


def _rsqrt_body(d0, d1, o):
    o[...] = lax.rsqrt(d0[...] + d1[...] + 1.0)


def _tc_rsqrt(deg):
    out = pl.pallas_call(
        _rsqrt_body,
        out_shape=jax.ShapeDtypeStruct((NP // D, D), jnp.float32),
    )(deg[0].reshape(NP // D, D), deg[1].reshape(NP // D, D))
    return out.reshape(NP)


@functools.cache
def _msg_kernel(ngroups):
    G = ngroups // (NC * NS)
    rows_per = NP // NS

    @functools.partial(
        pl.kernel,
        mesh=_mesh(),
        compiler_params=pltpu.CompilerParams(needs_layout_passes=False),
        out_type=jax.ShapeDtypeStruct((NC, NP, D), jnp.float32),
        scratch_types=[
            pltpu.VMEM((NP,), jnp.float32),
            pltpu.VMEM((GRP, CK), jnp.int32),
            pltpu.VMEM((GRP, CK), jnp.int32),
            pltpu.VMEM((GRP, CK), jnp.int32),
            pltpu.VMEM((GRP, CK), jnp.float32),
            pltpu.VMEM((GRP, CK), jnp.float32),
            [pltpu.VMEM((CK, D), jnp.float32) for _ in range(NBUF)],
            [pltpu.SemaphoreType.DMA for _ in range(2 * NBUF + 1)],
            pltpu.VMEM_SHARED((NP, D), jnp.float32),
        ],
    )
    def k(x2, src3, dst3, ew3, dinv_h, acc_h,
          dinv_v, srcA, srcB, dstg, ewg, normg, rows, sems, acc_sp):
        gsem = sems[:NBUF]
        ssem = sems[NBUF:2 * NBUF]
        psem = sems[2 * NBUF]
        c = lax.axis_index("c")
        s = lax.axis_index("s")
        base = (c * NS + s) * G
        pltpu.sync_copy(dinv_h, dinv_v)
        def zrow(r, carry):
            for kk in range(0, D, L):
                rows[0][r, pl.ds(kk, L)] = jnp.zeros((L,), jnp.float32)
            return carry

        lax.fori_loop(0, CK, zrow, 0)
        for t in range(rows_per // CK):
            pltpu.sync_copy(rows[0], acc_sp.at[pl.ds(s * rows_per + t * CK, CK)])

        pltpu.sync_copy(src3.at[base], srcA)
        for pj in range(4):
            pltpu.async_copy(x2.at[srcA.at[pj]], rows[pj], gsem[pj])
        plsc.subcore_barrier()

        def wait_scat(b):
            pltpu.make_async_copy(rows[b], acc_sp.at[dstg.at[0]],
                                  ssem[b]).wait()

        def main_g(g, carry):
            pass
            pltpu.sync_copy(dst3.at[base + g], dstg)
            pltpu.sync_copy(ew3.at[base + g], ewg)
            for j in range(GRP):
                for jj in range(0, CK, L):
                    nv = (ewg[j, pl.ds(jj, L)]
                          * plsc.load_gather(dinv_v, [srcA[j, pl.ds(jj, L)]])
                          * plsc.load_gather(dinv_v, [dstg[j, pl.ds(jj, L)]]))
                    normg[j, pl.ds(jj, L)] = nv

            for j in range(GRP):
                b = j % NBUF
                b2 = (j + 2) % NBUF
                if j == 2:
                    @pl.when(g + 1 < G)
                    def _():
                        pltpu.async_copy(src3.at[base + g + 1], srcB, psem)
                pltpu.make_async_copy(x2.at[srcA.at[j]], rows[b],
                                      gsem[b]).wait()
                if j == 4:
                    @pl.when(g + 1 < G)
                    def _():
                        pltpu.make_async_copy(src3.at[base + g + 1], srcB,
                                              psem).wait()
                if j < GRP - 4:
                    pltpu.async_copy(x2.at[srcA.at[j + 4]], rows[b], gsem[b])
                else:
                    @pl.when(g + 1 < G)
                    def _():
                        pltpu.async_copy(x2.at[srcB.at[j + 4 - GRP]], rows[b],
                                         gsem[b])
                if j == GRP - 1:
                    @pl.when(g + 1 < G)
                    def _():
                        for jc in range(GRP):
                            for jj in range(0, CK, L):
                                srcA[jc, pl.ds(jj, L)] = srcB[jc, pl.ds(jj, L)]
            return carry

        lax.fori_loop(0, G, main_g, 0)
        plsc.subcore_barrier()
        for t in range(rows_per // CK):
            pltpu.sync_copy(acc_sp.at[pl.ds(s * rows_per + t * CK, CK)], rows[0])
            pltpu.sync_copy(rows[0],
                            acc_h.at[c, pl.ds(s * rows_per + t * CK, CK)])

    return k


def _tc_body(a0, a1, x, dv, wc, bc, o):
    d2 = dv[...] * dv[...]
    px = a0[...] + a1[...] + d2 * x[...]
    y = jnp.dot(px, wc[...], preferred_element_type=jnp.float32,
                precision=lax.Precision.HIGHEST) + bc[...]
    z = jax.nn.sigmoid(y[:, :D])
    h = jnp.tanh(y[:, D:])
    o[...] = (1.0 - z) * h


def kernel(X, edge_index, edge_weight,
           W_xz, b_xz, W_hz, b_hz, W_xr, b_xr, W_hr, b_hr,
           W_xh, b_xh, W_hh, b_hh):
    n = X.shape[0]
    src = edge_index[0]
    dst = edge_index[1]
    e = src.shape[0]
    gsz = CK * GRP
    ngroups = -(-e // (gsz * NC * NS)) * NC * NS
    pad = ngroups * gsz - e
    src3 = jnp.concatenate([src, jnp.zeros((pad,), src.dtype)]).reshape(ngroups, GRP, CK)
    dst3 = jnp.concatenate([dst, jnp.zeros((pad,), dst.dtype)]).reshape(ngroups, GRP, CK)
    ew3 = jnp.concatenate(
        [edge_weight, jnp.zeros((pad,), edge_weight.dtype)]).reshape(ngroups, GRP, CK)
    xp = jnp.pad(X, ((0, NP - n), (0, 0)))

    deg = _deg_kernel(ngroups)(dst3, ew3)
    dinv = _tc_rsqrt(deg)
    acc = _msg_kernel(ngroups)(xp, src3, dst3, ew3, dinv)

    wc = jnp.concatenate([W_xz, W_xh], axis=1)
    bc = jnp.concatenate([b_xz + b_hz, b_xh + b_hh]).reshape(1, 2 * D)
    out = pl.pallas_call(
        _tc_body,
        grid=(NP // BLK,),
        in_specs=[
            pl.BlockSpec((BLK, D), lambda i: (i, 0)),
            pl.BlockSpec((BLK, D), lambda i: (i, 0)),
            pl.BlockSpec((BLK, D), lambda i: (i, 0)),
            pl.BlockSpec((BLK, 1), lambda i: (i, 0)),
            pl.BlockSpec((D, 2 * D), lambda i: (0, 0)),
            pl.BlockSpec((1, 2 * D), lambda i: (0, 0)),
        ],
        out_specs=pl.BlockSpec((BLK, D), lambda i: (i, 0)),
        out_shape=jax.ShapeDtypeStruct((NP, D), jnp.float32),
    )(acc[0], acc[1], xp, dinv.reshape(NP, 1), wc, bc)
    return out[:n]


# speedup vs baseline: 1.2013x; 1.2013x over previous
import functools

import jax
import jax.numpy as jnp
from jax import lax
from jax.experimental import pallas as pl
from jax.experimental.pallas import tpu as pltpu
from jax.experimental.pallas import tpu_sc as plsc

D = 128
NP = 10240
NC = 2
NS = 16
L = 16
CK = 64
GRP = 8
NBUF = 4
BLK = 128


def _mesh():
    return plsc.VectorSubcoreMesh(
        core_axis_name="c", subcore_axis_name="s", num_cores=NC, num_subcores=NS
    )


def _rsqrt_nr(x):
    i = plsc.bitcast(x, jnp.int32)
    i = jnp.int32(0x5F3759DF) - lax.shift_right_logical(i, 1)
    y = plsc.bitcast(i, jnp.float32)
    for _ in range(3):
        y = y * (1.5 - 0.5 * x * y * y)
    return y


@functools.cache
def _deg_kernel(ngroups):
    G = ngroups // NS
    slc = NP // (NC * NS)

    @functools.partial(
        pl.kernel,
        mesh=_mesh(),
        compiler_params=pltpu.CompilerParams(needs_layout_passes=False),
        out_type=jax.ShapeDtypeStruct((NP,), jnp.float32),
        scratch_types=[
            pltpu.VMEM((G, GRP, CK), jnp.int32),
            pltpu.VMEM((G, GRP, CK), jnp.float32),
            pltpu.VMEM((NP // NS,), jnp.float32),
            pltpu.VMEM_SHARED((NP,), jnp.float32),
            pltpu.SemaphoreType.DMA,
        ],
    )
    def k(dst3, ew3, dinv, dstb3, ewb3, zbuf, deg_sp, ssem):
        c = lax.axis_index("c")
        s = lax.axis_index("s")
        w = c * NS + s
        for j in range(0, NP // NS, L):
            zbuf[pl.ds(j, L)] = jnp.zeros((L,), jnp.float32)
        pltpu.sync_copy(zbuf, deg_sp.at[pl.ds(s * (NP // NS), NP // NS)])
        pltpu.sync_copy(dst3.at[pl.ds(s * G, G)], dstb3)
        pltpu.sync_copy(ew3.at[pl.ds(s * G, G)], ewb3)
        plsc.subcore_barrier()

        def grp_body(g, carry):
            for j in range(GRP):
                pltpu.async_copy(ewb3.at[g, j], deg_sp.at[dstb3.at[g, j]],
                                 ssem, add=True)
            for j in range(GRP):
                pltpu.make_async_copy(ewb3.at[g, j],
                                      deg_sp.at[dstb3.at[g, j]], ssem).wait()
            return carry

        lax.fori_loop(0, G, grp_body, 0)
        plsc.subcore_barrier()
        pltpu.sync_copy(deg_sp.at[pl.ds(w * slc, slc)], zbuf.at[pl.ds(0, slc)])
        for j in range(0, slc, L):
            zbuf[pl.ds(j, L)] = _rsqrt_nr(zbuf[pl.ds(j, L)] + 1.0)
        pltpu.sync_copy(zbuf.at[pl.ds(0, slc)], dinv.at[pl.ds(w * slc, slc)])

    return k


@functools.cache
def _msg_kernel(ngroups):
    G = ngroups // (NC * NS)
    rows_per = NP // NS

    @functools.partial(
        pl.kernel,
        mesh=_mesh(),
        compiler_params=pltpu.CompilerParams(needs_layout_passes=False),
        out_type=jax.ShapeDtypeStruct((NC, NP, D), jnp.float32),
        scratch_types=[
            pltpu.VMEM((NP,), jnp.float32),
            pltpu.VMEM((GRP, CK), jnp.int32),
            pltpu.VMEM((GRP, CK), jnp.int32),
            pltpu.VMEM((GRP, CK), jnp.int32),
            pltpu.VMEM((GRP, CK), jnp.float32),
            pltpu.VMEM((GRP, CK), jnp.float32),
            [pltpu.VMEM((CK, D), jnp.float32) for _ in range(NBUF)],
            [pltpu.SemaphoreType.DMA for _ in range(2 * NBUF + 1)],
            pltpu.VMEM_SHARED((NP, D), jnp.float32),
        ],
    )
    def k(x2, src3, dst3, ew3, dinv_h, acc_h,
          dinv_v, srcA, srcB, dstg, ewg, normg, rows, sems, acc_sp):
        gsem = sems[:NBUF]
        ssem = sems[NBUF:2 * NBUF]
        psem = sems[2 * NBUF]
        c = lax.axis_index("c")
        s = lax.axis_index("s")
        base = (c * NS + s) * G
        pltpu.sync_copy(dinv_h, dinv_v)
        def zrow(r, carry):
            for kk in range(0, D, L):
                rows[0][r, pl.ds(kk, L)] = jnp.zeros((L,), jnp.float32)
            return carry

        lax.fori_loop(0, CK, zrow, 0)
        for t in range(rows_per // CK):
            pltpu.sync_copy(rows[0], acc_sp.at[pl.ds(s * rows_per + t * CK, CK)])

        pltpu.sync_copy(src3.at[base], srcA)
        pltpu.async_copy(x2.at[srcA.at[0]], rows[0], gsem[0])
        pltpu.async_copy(x2.at[srcA.at[1]], rows[1], gsem[1])
        plsc.subcore_barrier()

        def wait_scat(b):
            pltpu.make_async_copy(rows[b], acc_sp.at[dstg.at[0]],
                                  ssem[b]).wait()

        def main_g(g, carry):
            @pl.when(g > 0)
            def _():
                wait_scat(2)
                wait_scat(3)
            pltpu.sync_copy(dst3.at[base + g], dstg)
            pltpu.sync_copy(ew3.at[base + g], ewg)
            for j in range(GRP):
                for jj in range(0, CK, L):
                    nv = (ewg[j, pl.ds(jj, L)]
                          * plsc.load_gather(dinv_v, [srcA[j, pl.ds(jj, L)]])
                          * plsc.load_gather(dinv_v, [dstg[j, pl.ds(jj, L)]]))
                    normg[j, pl.ds(jj, L)] = nv

            for j in range(GRP):
                b = j % NBUF
                b2 = (j + 2) % NBUF
                if j == 2:
                    @pl.when(g + 1 < G)
                    def _():
                        pltpu.async_copy(src3.at[base + g + 1], srcB, psem)
                pltpu.make_async_copy(x2.at[srcA.at[j]], rows[b],
                                      gsem[b]).wait()
                jj_ = jnp.full((L,), j, jnp.int32)

                def erow(e2, cc):
                    for u in range(2):
                        e = e2 * 2 + u
                        ns_ = plsc.load_gather(
                            normg, [jj_, jnp.broadcast_to(e, (L,))])
                        for kk in range(0, D, L):
                            rows[b][e, pl.ds(kk, L)] = (
                                rows[b][e, pl.ds(kk, L)] * ns_)
                    return cc

                lax.fori_loop(0, CK // 2, erow, 0)
                if j >= 2:
                    wait_scat(b2)
                if j < GRP - 2:
                    pltpu.async_copy(x2.at[srcA.at[j + 2]], rows[b2], gsem[b2])
                else:
                    @pl.when(g + 1 < G)
                    def _():
                        pltpu.async_copy(x2.at[srcB.at[j + 2 - GRP]], rows[b2],
                                         gsem[b2])
                pltpu.async_copy(rows[b], acc_sp.at[dstg.at[j]], ssem[b],
                                 add=True)
                if j == GRP - 1:
                    @pl.when(g + 1 < G)
                    def _():
                        pltpu.make_async_copy(src3.at[base + g + 1], srcB,
                                              psem).wait()
                        for jc in range(GRP):
                            for jj in range(0, CK, L):
                                srcA[jc, pl.ds(jj, L)] = srcB[jc, pl.ds(jj, L)]
            return carry

        lax.fori_loop(0, G, main_g, 0)
        wait_scat(2)
        wait_scat(3)
        plsc.subcore_barrier()
        for t in range(rows_per // CK):
            pltpu.sync_copy(acc_sp.at[pl.ds(s * rows_per + t * CK, CK)], rows[0])
            pltpu.sync_copy(rows[0],
                            acc_h.at[c, pl.ds(s * rows_per + t * CK, CK)])

    return k


def _tc_body(a0, a1, x, dv, wc, bc, o):
    d2 = dv[...] * dv[...]
    px = a0[...] + a1[...] + d2 * x[...]
    y = jnp.dot(px, wc[...], preferred_element_type=jnp.float32,
                precision=lax.Precision.HIGHEST) + bc[...]
    z = jax.nn.sigmoid(y[:, :D])
    h = jnp.tanh(y[:, D:])
    o[...] = (1.0 - z) * h


def kernel(X, edge_index, edge_weight,
           W_xz, b_xz, W_hz, b_hz, W_xr, b_xr, W_hr, b_hr,
           W_xh, b_xh, W_hh, b_hh):
    n = X.shape[0]
    src = edge_index[0]
    dst = edge_index[1]
    e = src.shape[0]
    gsz = CK * GRP
    ngroups = -(-e // (gsz * NC * NS)) * NC * NS
    pad = ngroups * gsz - e
    src3 = jnp.concatenate([src, jnp.zeros((pad,), src.dtype)]).reshape(ngroups, GRP, CK)
    dst3 = jnp.concatenate([dst, jnp.zeros((pad,), dst.dtype)]).reshape(ngroups, GRP, CK)
    ew3 = jnp.concatenate(
        [edge_weight, jnp.zeros((pad,), edge_weight.dtype)]).reshape(ngroups, GRP, CK)
    xp = jnp.pad(X, ((0, NP - n), (0, 0)))

    dinv = _deg_kernel(ngroups)(dst3, ew3)
    acc = _msg_kernel(ngroups)(xp, src3, dst3, ew3, dinv)

    wc = jnp.concatenate([W_xz, W_xh], axis=1)
    bc = jnp.concatenate([b_xz + b_hz, b_xh + b_hh]).reshape(1, 2 * D)
    out = pl.pallas_call(
        _tc_body,
        grid=(NP // BLK,),
        in_specs=[
            pl.BlockSpec((BLK, D), lambda i: (i, 0)),
            pl.BlockSpec((BLK, D), lambda i: (i, 0)),
            pl.BlockSpec((BLK, D), lambda i: (i, 0)),
            pl.BlockSpec((BLK, 1), lambda i: (i, 0)),
            pl.BlockSpec((D, 2 * D), lambda i: (0, 0)),
            pl.BlockSpec((1, 2 * D), lambda i: (0, 0)),
        ],
        out_specs=pl.BlockSpec((BLK, D), lambda i: (i, 0)),
        out_shape=jax.ShapeDtypeStruct((NP, D), jnp.float32),
    )(acc[0], acc[1], xp, dinv.reshape(NP, 1), wc, bc)
    return out[:n]
